# baseline (speedup 1.0000x reference)
import functools
import jax
import jax.numpy as jnp
from jax import lax
from jax.experimental import pallas as pl
from jax.experimental.pallas import tpu as pltpu
from jax.experimental.pallas import tpu_sc as plsc

DIM = 64
SEQ = 200
BATCH = 1024
NROWS = BATCH * SEQ
NW = 32
CHUNK = 100
CH_PW = NROWS // CHUNK // NW
NPAIR = CH_PW // 2
SCALE = 8.0

_mesh = plsc.VectorSubcoreMesh(core_axis_name="c", subcore_axis_name="s")


@functools.partial(
    pl.kernel,
    mesh=_mesh,
    compiler_params=pltpu.CompilerParams(use_tc_tiling_on_sc=False),
    out_type=jax.ShapeDtypeStruct((NROWS // CHUNK, CHUNK, DIM), jnp.float32),
    scratch_types=[
        pltpu.VMEM((CH_PW, CHUNK), jnp.int32),
        pltpu.VMEM((SEQ, DIM), jnp.float32),
        pltpu.VMEM((CHUNK, DIM), jnp.float32),
        pltpu.VMEM((CHUNK, DIM), jnp.float32),
        pltpu.SemaphoreType.DMA,
        pltpu.SemaphoreType.DMA,
    ],
)
def _emb_kernel(idx_hbm, pe_hbm, tab_hbm, out_hbm, idx_v, pe_v, buf0, buf1,
                s0, s1):
    wid = lax.axis_index("s") * 2 + lax.axis_index("c")
    cbase = wid * CH_PW

    pltpu.sync_copy(idx_hbm.at[pl.ds(cbase, CH_PW)], idx_v)
    pltpu.sync_copy(pe_hbm, pe_v)

    pltpu.async_copy(tab_hbm.at[idx_v.at[0]], buf0, s0)
    pltpu.async_copy(tab_hbm.at[idx_v.at[1]], buf1, s1)

    def pair(c2, carry):
        for k, (buf, sem) in enumerate(((buf0, s0), (buf1, s1))):
            c = c2 * 2 + k
            pltpu.make_async_copy(tab_hbm.at[idx_v.at[0]], buf, sem).wait()

            def body(r, _):
                pr = k * CHUNK + r
                for j in range(DIM // 16):
                    sl = pl.ds(j * 16, 16)
                    buf[r, sl] = buf[r, sl] * SCALE + pe_v[pr, sl]
                return 0

            lax.fori_loop(0, CHUNK, body, 0)
            pltpu.sync_copy(buf, out_hbm.at[cbase + c])

            @pl.when(c2 < NPAIR - 1)
            def _():
                pltpu.async_copy(tab_hbm.at[idx_v.at[c + 2]], buf, sem)

        return carry

    lax.fori_loop(0, NPAIR, pair, 0)


def kernel(x, embed_weight, pe):
    idx = x.reshape(NROWS // CHUNK, CHUNK).astype(jnp.int32)
    pe2 = pe[0, :SEQ].astype(jnp.float32)
    out = _emb_kernel(idx, pe2, embed_weight)
    return out.reshape(BATCH, SEQ, DIM)


# speedup vs baseline: 1.0005x; 1.0005x over previous
import functools
import jax
import jax.numpy as jnp
from jax import lax
from jax.experimental import pallas as pl
from jax.experimental.pallas import tpu as pltpu
from jax.experimental.pallas import tpu_sc as plsc

DIM = 64
SEQ = 200
BATCH = 1024
NW = 32
B_PW = BATCH // NW
CHUNK = SEQ // 2
SCALE = 8.0

_mesh = plsc.VectorSubcoreMesh(core_axis_name="c", subcore_axis_name="s")


@functools.partial(
    pl.kernel,
    mesh=_mesh,
    compiler_params=pltpu.CompilerParams(use_tc_tiling_on_sc=False),
    out_type=jax.ShapeDtypeStruct((BATCH, SEQ, DIM), jnp.float32),
    scratch_types=[
        pltpu.VMEM((2 * B_PW, CHUNK), jnp.int32),
        pltpu.VMEM((SEQ, DIM), jnp.float32),
        pltpu.VMEM((CHUNK, DIM), jnp.float32),
        pltpu.VMEM((CHUNK, DIM), jnp.float32),
        pltpu.SemaphoreType.DMA,
        pltpu.SemaphoreType.DMA,
    ],
)
def _emb_kernel(idx_hbm, pe_hbm, tab_hbm, out_hbm, idx_v, pe_v, buf0, buf1,
                s0, s1):
    wid = lax.axis_index("s") * 2 + lax.axis_index("c")
    b0 = wid * B_PW
    cbase = wid * 2 * B_PW

    pltpu.sync_copy(idx_hbm.at[pl.ds(cbase, 2 * B_PW)], idx_v)
    pltpu.sync_copy(pe_hbm, pe_v)

    pltpu.async_copy(tab_hbm.at[idx_v.at[0]], buf0, s0)
    pltpu.async_copy(tab_hbm.at[idx_v.at[1]], buf1, s1)

    def seq_step(bb, carry):
        for k, (buf, sem) in enumerate(((buf0, s0), (buf1, s1))):
            pltpu.make_async_copy(tab_hbm.at[idx_v.at[0]], buf, sem).wait()

            def body(r, _):
                pr = k * CHUNK + r
                for j in range(DIM // 16):
                    sl = pl.ds(j * 16, 16)
                    buf[r, sl] = buf[r, sl] * SCALE + pe_v[pr, sl]
                return 0

            lax.fori_loop(0, CHUNK, body, 0)
            pltpu.sync_copy(
                buf, out_hbm.at[b0 + bb, pl.ds(k * CHUNK, CHUNK)])

            @pl.when(bb < B_PW - 1)
            def _():
                pltpu.async_copy(
                    tab_hbm.at[idx_v.at[2 * bb + 2 + k]], buf, sem)

        return carry

    lax.fori_loop(0, B_PW, seq_step, 0)


def kernel(x, embed_weight, pe):
    idx = x.reshape(BATCH * SEQ // CHUNK, CHUNK).astype(jnp.int32)
    pe2 = pe[0, :SEQ].astype(jnp.float32)
    return _emb_kernel(idx, pe2, embed_weight)
